# baseline (speedup 1.0000x reference)
import math

import functools

import jax
import jax.numpy as jnp
from jax import lax
from jax.experimental import pallas as pl

_MM_DTYPE = jnp.bfloat16


def _dot(a, b):
    return jnp.dot(a.astype(_MM_DTYPE), b.astype(_MM_DTYPE),
                   preferred_element_type=jnp.float32)


def _gcn(at, x, w, b, relu):
    deg = jnp.sum(at, axis=1, keepdims=True) + 2.0
    dinv = lax.rsqrt(deg)
    y = _dot(x, w) * dinv
    out = (_dot(at, y) + 2.0 * y) * dinv + b
    return jnp.maximum(out, 0.0) if relu else out


def _score(h, p):
    inv_norm = lax.rsqrt(jnp.sum(p * p))
    return jnp.tanh(jnp.sum(h * p, axis=1, keepdims=True) * inv_norm)


def _offdiag_aug(r, c):
    m = _dot(r, c)
    k = m.shape[0]
    eye = (lax.broadcasted_iota(jnp.int32, (k, k), 0) ==
           lax.broadcasted_iota(jnp.int32, (k, k), 1)).astype(m.dtype)
    return m * (1.0 - eye)


def _conv_score_body(at_ref, x_ref, w_ref, b_ref, p_ref, o_ref, s_ref):
    h = _gcn(at_ref[...], x_ref[...], w_ref[...], b_ref[...], relu=True)
    o_ref[...] = h
    s_ref[...] = jnp.broadcast_to(_score(h, p_ref[...]), s_ref.shape)


def _down_mid_body(r_ref, c_ref, x_ref, w_ref, b_ref, p_ref,
                   a_ref, o_ref, s_ref):
    atn = _offdiag_aug(r_ref[...], c_ref[...])
    a_ref[...] = atn
    h = _gcn(atn, x_ref[...], w_ref[...], b_ref[...], relu=True)
    o_ref[...] = h
    s_ref[...] = jnp.broadcast_to(_score(h, p_ref[...]), s_ref.shape)


def _down_last_body(r_ref, c_ref, x_ref, w_ref, b_ref, o_ref):
    atn = _offdiag_aug(r_ref[...], c_ref[...])
    o_ref[...] = _gcn(atn, x_ref[...], w_ref[...], b_ref[...], relu=True)


def _conv_body(at_ref, x_ref, w_ref, b_ref, o_ref, *, relu):
    o_ref[...] = _gcn(at_ref[...], x_ref[...], w_ref[...], b_ref[...], relu)


def _full(shape):
    return pl.BlockSpec(shape, lambda i: (0,) * len(shape))


def _conv_score(at, x, w, b, p):
    n, co = at.shape[0], w.shape[1]
    return pl.pallas_call(
        _conv_score_body,
        out_shape=(jax.ShapeDtypeStruct((n, co), jnp.float32),
                   jax.ShapeDtypeStruct((n, 128), jnp.float32)),
        grid=(1,),
        in_specs=[_full(at.shape), _full(x.shape), _full(w.shape),
                  _full(b.shape), _full(p.shape)],
        out_specs=(_full((n, co)), _full((n, 128))),
    )(at, x, w, b, p)


def _down_mid(r, c, x, w, b, p):
    k, co = r.shape[0], w.shape[1]
    return pl.pallas_call(
        _down_mid_body,
        out_shape=(jax.ShapeDtypeStruct((k, k), jnp.float32),
                   jax.ShapeDtypeStruct((k, co), jnp.float32),
                   jax.ShapeDtypeStruct((k, 128), jnp.float32)),
        grid=(1,),
        in_specs=[_full(r.shape), _full(c.shape), _full(x.shape),
                  _full(w.shape), _full(b.shape), _full(p.shape)],
        out_specs=(_full((k, k)), _full((k, co)), _full((k, 128))),
    )(r, c, x, w, b, p)


def _down_last(r, c, x, w, b):
    k, co = r.shape[0], w.shape[1]
    return pl.pallas_call(
        _down_last_body,
        out_shape=jax.ShapeDtypeStruct((k, co), jnp.float32),
        grid=(1,),
        in_specs=[_full(r.shape), _full(c.shape), _full(x.shape),
                  _full(w.shape), _full(b.shape)],
        out_specs=_full((k, co)),
    )(r, c, x, w, b)


def _conv(at, x, w, b, relu):
    n, co = at.shape[0], w.shape[1]
    body = functools.partial(_conv_body, relu=relu)
    return pl.pallas_call(
        body,
        out_shape=jax.ShapeDtypeStruct((n, co), jnp.float32),
        grid=(1,),
        in_specs=[_full(at.shape), _full(x.shape), _full(w.shape),
                  _full(b.shape)],
        out_specs=_full((n, co)),
    )(at, x, w, b)


def _pool(at, x, s, kk):
    _, perm = lax.top_k(s, kk)
    n = at.shape[0]
    bmat = at + jnp.eye(n, dtype=at.dtype)
    r = bmat[perm]
    c = jnp.take(bmat, perm, axis=1)
    xg = x[perm] * s[perm][:, None]
    return xg, r, c, perm


def kernel(x, adj, down_w_0, down_w_1, down_w_2, down_w_3,
           down_b_0, down_b_1, down_b_2, down_b_3,
           pool_p_0, pool_p_1, pool_p_2,
           up_w_0, up_w_1, up_w_2,
           up_b_0, up_b_1, up_b_2):
    at0 = adj.T
    n = adj.shape[0]
    k1 = int(math.ceil(0.5 * n))
    k2 = int(math.ceil(0.5 * k1))
    k3 = int(math.ceil(0.5 * k2))

    x0, s0m = _conv_score(at0, x, down_w_0, down_b_0, pool_p_0)
    s0 = s0m[:, 0]

    xg, r, c, perm1 = _pool(at0, x0, s0, k1)
    at1, x1, s1m = _down_mid(r, c, xg, down_w_1, down_b_1, pool_p_1)
    s1 = s1m[:, 0]

    xg, r, c, perm2 = _pool(at1, x1, s1, k2)
    at2, x2, s2m = _down_mid(r, c, xg, down_w_2, down_b_2, pool_p_2)
    s2 = s2m[:, 0]

    xg, r, c, perm3 = _pool(at2, x2, s2, k3)
    x3 = _down_last(r, c, xg, down_w_3, down_b_3)

    u = x2.at[perm3].add(x3)
    u = _conv(at2, u, up_w_0, up_b_0, relu=True)
    u = x1.at[perm2].add(u)
    u = _conv(at1, u, up_w_1, up_b_1, relu=True)
    u = x0.at[perm1].add(u)
    return _conv(at0, u, up_w_2, up_b_2, relu=False)


# speedup vs baseline: 2.6334x; 2.6334x over previous
import math

import functools

import jax
import jax.numpy as jnp
from jax import lax
from jax.experimental import pallas as pl

_MM_DTYPE = jnp.bfloat16


def _dot(a, b):
    return jnp.dot(a.astype(_MM_DTYPE), b.astype(_MM_DTYPE),
                   preferred_element_type=jnp.float32)


def _gcn(at, x, w, b, relu):
    deg = jnp.sum(at, axis=1, keepdims=True) + 2.0
    dinv = lax.rsqrt(deg)
    y = _dot(x, w) * dinv
    out = (_dot(at, y) + 2.0 * y) * dinv + b
    return jnp.maximum(out, 0.0) if relu else out


def _score(h, p):
    inv_norm = lax.rsqrt(jnp.sum(p * p))
    return jnp.tanh(jnp.sum(h * p, axis=1, keepdims=True) * inv_norm)


def _offdiag_aug(r, c):
    m = _dot(r, c)
    k = m.shape[0]
    eye = (lax.broadcasted_iota(jnp.int32, (k, k), 0) ==
           lax.broadcasted_iota(jnp.int32, (k, k), 1)).astype(m.dtype)
    return m * (1.0 - eye)


def _conv_score_body(at_ref, x_ref, w_ref, b_ref, p_ref, o_ref, s_ref):
    h = _gcn(at_ref[...], x_ref[...], w_ref[...], b_ref[...], relu=True)
    o_ref[...] = h
    s_ref[...] = jnp.broadcast_to(_score(h, p_ref[...]), s_ref.shape)


def _down_mid_body(r_ref, c_ref, x_ref, w_ref, b_ref, p_ref,
                   a_ref, o_ref, s_ref):
    atn = _offdiag_aug(r_ref[...], c_ref[...])
    a_ref[...] = atn
    h = _gcn(atn, x_ref[...], w_ref[...], b_ref[...], relu=True)
    o_ref[...] = h
    s_ref[...] = jnp.broadcast_to(_score(h, p_ref[...]), s_ref.shape)


def _down_last_body(r_ref, c_ref, x_ref, w_ref, b_ref, o_ref):
    atn = _offdiag_aug(r_ref[...], c_ref[...])
    o_ref[...] = _gcn(atn, x_ref[...], w_ref[...], b_ref[...], relu=True)


def _up_conv_body(perm_ref, xu_ref, res_ref, at_ref, w_ref, b_ref, o_ref, *,
                  relu):
    n = res_ref.shape[0]
    kk = perm_ref.shape[1]
    pm = perm_ref[...]
    ri = lax.broadcasted_iota(jnp.int32, (n, kk), 0).astype(jnp.float32)
    pt = (pm == ri).astype(jnp.float32)
    u = res_ref[...] + jnp.dot(pt, xu_ref[...],
                               preferred_element_type=jnp.float32)
    o_ref[...] = _gcn(at_ref[...], u, w_ref[...], b_ref[...], relu)


def _full(shape):
    return pl.BlockSpec(shape, lambda i: (0,) * len(shape))


def _conv_score(at, x, w, b, p):
    n, co = at.shape[0], w.shape[1]
    return pl.pallas_call(
        _conv_score_body,
        out_shape=(jax.ShapeDtypeStruct((n, co), jnp.float32),
                   jax.ShapeDtypeStruct((n, 128), jnp.float32)),
        grid=(1,),
        in_specs=[_full(at.shape), _full(x.shape), _full(w.shape),
                  _full(b.shape), _full(p.shape)],
        out_specs=(_full((n, co)), _full((n, 128))),
    )(at, x, w, b, p)


def _down_mid(r, c, x, w, b, p):
    k, co = r.shape[0], w.shape[1]
    return pl.pallas_call(
        _down_mid_body,
        out_shape=(jax.ShapeDtypeStruct((k, k), jnp.float32),
                   jax.ShapeDtypeStruct((k, co), jnp.float32),
                   jax.ShapeDtypeStruct((k, 128), jnp.float32)),
        grid=(1,),
        in_specs=[_full(r.shape), _full(c.shape), _full(x.shape),
                  _full(w.shape), _full(b.shape), _full(p.shape)],
        out_specs=(_full((k, k)), _full((k, co)), _full((k, 128))),
    )(r, c, x, w, b, p)


def _down_last(r, c, x, w, b):
    k, co = r.shape[0], w.shape[1]
    return pl.pallas_call(
        _down_last_body,
        out_shape=jax.ShapeDtypeStruct((k, co), jnp.float32),
        grid=(1,),
        in_specs=[_full(r.shape), _full(c.shape), _full(x.shape),
                  _full(w.shape), _full(b.shape)],
        out_specs=_full((k, co)),
    )(r, c, x, w, b)


def _up_conv(perm_f, xu, res, at, w, b, relu):
    n, co = at.shape[0], w.shape[1]
    body = functools.partial(_up_conv_body, relu=relu)
    return pl.pallas_call(
        body,
        out_shape=jax.ShapeDtypeStruct((n, co), jnp.float32),
        grid=(1,),
        in_specs=[_full(perm_f.shape), _full(xu.shape), _full(res.shape),
                  _full(at.shape), _full(w.shape), _full(b.shape)],
        out_specs=_full((n, co)),
    )(perm_f, xu, res, at, w, b)


def _pool(at, x, s, kk):
    _, perm = lax.top_k(s, kk)
    n = at.shape[0]
    bmat = at + jnp.eye(n, dtype=at.dtype)
    r = bmat[perm]
    c = jnp.take(bmat, perm, axis=1)
    xg = x[perm] * s[perm][:, None]
    return xg, r, c, perm


def kernel(x, adj, down_w_0, down_w_1, down_w_2, down_w_3,
           down_b_0, down_b_1, down_b_2, down_b_3,
           pool_p_0, pool_p_1, pool_p_2,
           up_w_0, up_w_1, up_w_2,
           up_b_0, up_b_1, up_b_2):
    at0 = adj.T
    n = adj.shape[0]
    k1 = int(math.ceil(0.5 * n))
    k2 = int(math.ceil(0.5 * k1))
    k3 = int(math.ceil(0.5 * k2))

    x0, s0m = _conv_score(at0, x, down_w_0, down_b_0, pool_p_0)
    s0 = s0m[:, 0]

    xg, r, c, perm1 = _pool(at0, x0, s0, k1)
    at1, x1, s1m = _down_mid(r, c, xg, down_w_1, down_b_1, pool_p_1)
    s1 = s1m[:, 0]

    xg, r, c, perm2 = _pool(at1, x1, s1, k2)
    at2, x2, s2m = _down_mid(r, c, xg, down_w_2, down_b_2, pool_p_2)
    s2 = s2m[:, 0]

    xg, r, c, perm3 = _pool(at2, x2, s2, k3)
    x3 = _down_last(r, c, xg, down_w_3, down_b_3)

    p3f = perm3.astype(jnp.float32)[None, :]
    u = _up_conv(p3f, x3, x2, at2, up_w_0, up_b_0, relu=True)
    p2f = perm2.astype(jnp.float32)[None, :]
    u = _up_conv(p2f, u, x1, at1, up_w_1, up_b_1, relu=True)
    p1f = perm1.astype(jnp.float32)[None, :]
    return _up_conv(p1f, u, x0, at0, up_w_2, up_b_2, relu=False)


# speedup vs baseline: 4.0373x; 1.5331x over previous
import math

import functools

import jax
import jax.numpy as jnp
from jax import lax
from jax.experimental import pallas as pl

_MM_DTYPE = jnp.bfloat16


def _dot(a, b):
    return jnp.dot(a.astype(_MM_DTYPE), b.astype(_MM_DTYPE),
                   preferred_element_type=jnp.float32)


def _dotf(a, b):
    return jnp.dot(a, b, preferred_element_type=jnp.float32)


def _gcn(at, x, w, b, relu):
    deg = jnp.sum(at, axis=1, keepdims=True) + 2.0
    dinv = lax.rsqrt(deg)
    y = _dot(x, w) * dinv
    out = (_dot(at, y) + 2.0 * y) * dinv + b
    return jnp.maximum(out, 0.0) if relu else out


def _score(h, p):
    inv_norm = lax.rsqrt(jnp.sum(p * p))
    return jnp.tanh(jnp.sum(h * p, axis=1, keepdims=True) * inv_norm)


def _onehots(pr, pc, n):
    kk = pr.shape[1]
    ci = lax.broadcasted_iota(jnp.int32, (kk, n), 1).astype(jnp.float32)
    ri = lax.broadcasted_iota(jnp.int32, (n, kk), 0).astype(jnp.float32)
    p_sel = (pc == ci).astype(jnp.float32)
    pt_sel = (pr == ri).astype(jnp.float32)
    return p_sel, pt_sel


def _pooled_adj(at, p_sel, pt_sel, exact):
    d = _dotf if exact else _dot
    r = d(p_sel, at) + p_sel
    c = d(at, pt_sel) + pt_sel
    m = _dot(r, c)
    kk = m.shape[0]
    eye = (lax.broadcasted_iota(jnp.int32, (kk, kk), 0) ==
           lax.broadcasted_iota(jnp.int32, (kk, kk), 1)).astype(m.dtype)
    return m * (1.0 - eye)


def _gather_x(p_sel, x, s):
    sg = _dotf(p_sel, s[:, :1])
    return _dotf(p_sel, x) * sg


def _conv_score_body(at_ref, x_ref, w_ref, b_ref, p_ref, o_ref, s_ref):
    h = _gcn(at_ref[...], x_ref[...], w_ref[...], b_ref[...], relu=True)
    o_ref[...] = h
    s_ref[...] = jnp.broadcast_to(_score(h, p_ref[...]), s_ref.shape)


def _down_mid_body(a_ref, x_ref, s_ref, pr_ref, pc_ref, w_ref, b_ref, p_ref,
                   ao_ref, o_ref, so_ref, *, exact):
    n = a_ref.shape[0]
    p_sel, pt_sel = _onehots(pr_ref[...], pc_ref[...], n)
    atn = _pooled_adj(a_ref[...], p_sel, pt_sel, exact)
    ao_ref[...] = atn
    xg = _gather_x(p_sel, x_ref[...], s_ref[...])
    h = _gcn(atn, xg, w_ref[...], b_ref[...], relu=True)
    o_ref[...] = h
    so_ref[...] = jnp.broadcast_to(_score(h, p_ref[...]), so_ref.shape)


def _bottom_body(a_ref, x_ref, s_ref, pr_ref, pc_ref, wd_ref, bd_ref,
                 wu_ref, bu_ref, o_ref):
    n = a_ref.shape[0]
    p_sel, pt_sel = _onehots(pr_ref[...], pc_ref[...], n)
    at3 = _pooled_adj(a_ref[...], p_sel, pt_sel, exact=True)
    xg = _gather_x(p_sel, x_ref[...], s_ref[...])
    x3 = _gcn(at3, xg, wd_ref[...], bd_ref[...], relu=True)
    u = x_ref[...] + _dotf(pt_sel, x3)
    o_ref[...] = _gcn(a_ref[...], u, wu_ref[...], bu_ref[...], relu=True)


def _up_conv_body(pr_ref, xu_ref, res_ref, at_ref, w_ref, b_ref, o_ref, *,
                  relu):
    n = res_ref.shape[0]
    kk = pr_ref.shape[1]
    ri = lax.broadcasted_iota(jnp.int32, (n, kk), 0).astype(jnp.float32)
    pt_sel = (pr_ref[...] == ri).astype(jnp.float32)
    u = res_ref[...] + _dotf(pt_sel, xu_ref[...])
    o_ref[...] = _gcn(at_ref[...], u, w_ref[...], b_ref[...], relu)


def _full(shape):
    return pl.BlockSpec(shape, lambda i: (0,) * len(shape))


def _call(body, ins, outs):
    return pl.pallas_call(
        body,
        out_shape=outs,
        grid=(1,),
        in_specs=[_full(a.shape) for a in ins],
        out_specs=jax.tree.map(lambda s: _full(s.shape), outs),
    )(*ins)


def _conv_score(at, x, w, b, p):
    n, co = at.shape[0], w.shape[1]
    return _call(_conv_score_body, (at, x, w, b, p),
                 (jax.ShapeDtypeStruct((n, co), jnp.float32),
                  jax.ShapeDtypeStruct((n, 128), jnp.float32)))


def _down_mid(at, x, s, pr, pc, w, b, p, exact):
    kk, co = pr.shape[1], w.shape[1]
    body = functools.partial(_down_mid_body, exact=exact)
    return _call(body, (at, x, s, pr, pc, w, b, p),
                 (jax.ShapeDtypeStruct((kk, kk), jnp.float32),
                  jax.ShapeDtypeStruct((kk, co), jnp.float32),
                  jax.ShapeDtypeStruct((kk, 128), jnp.float32)))


def _bottom(at, x, s, pr, pc, wd, bd, wu, bu):
    n, co = at.shape[0], wu.shape[1]
    return _call(_bottom_body, (at, x, s, pr, pc, wd, bd, wu, bu),
                 jax.ShapeDtypeStruct((n, co), jnp.float32))


def _up_conv(pr, xu, res, at, w, b, relu):
    n, co = at.shape[0], w.shape[1]
    body = functools.partial(_up_conv_body, relu=relu)
    return _call(body, (pr, xu, res, at, w, b),
                 jax.ShapeDtypeStruct((n, co), jnp.float32))


def _perm_args(s, kk):
    _, perm = lax.top_k(s, kk)
    pf = perm.astype(jnp.float32)
    return pf[None, :], pf[:, None]


def kernel(x, adj, down_w_0, down_w_1, down_w_2, down_w_3,
           down_b_0, down_b_1, down_b_2, down_b_3,
           pool_p_0, pool_p_1, pool_p_2,
           up_w_0, up_w_1, up_w_2,
           up_b_0, up_b_1, up_b_2):
    at0 = adj.T
    n = adj.shape[0]
    k1 = int(math.ceil(0.5 * n))
    k2 = int(math.ceil(0.5 * k1))
    k3 = int(math.ceil(0.5 * k2))

    x0, s0 = _conv_score(at0, x, down_w_0, down_b_0, pool_p_0)
    pr1, pc1 = _perm_args(s0[:, 0], k1)
    at1, x1, s1 = _down_mid(at0, x0, s0, pr1, pc1, down_w_1, down_b_1,
                            pool_p_1, exact=False)
    pr2, pc2 = _perm_args(s1[:, 0], k2)
    at2, x2, s2 = _down_mid(at1, x1, s1, pr2, pc2, down_w_2, down_b_2,
                            pool_p_2, exact=False)
    pr3, pc3 = _perm_args(s2[:, 0], k3)
    u = _bottom(at2, x2, s2, pr3, pc3, down_w_3, down_b_3, up_w_0, up_b_0)
    u = _up_conv(pr2, u, x1, at1, up_w_1, up_b_1, relu=True)
    return _up_conv(pr1, u, x0, at0, up_w_2, up_b_2, relu=False)


# speedup vs baseline: 4.2787x; 1.0598x over previous
import math

import functools

import jax
import jax.numpy as jnp
from jax import lax
from jax.experimental import pallas as pl

_MM_DTYPE = jnp.bfloat16
_DIMS_T = (((0,), (0,)), ((), ()))


def _dot(a, b):
    return jnp.dot(a.astype(_MM_DTYPE), b.astype(_MM_DTYPE),
                   preferred_element_type=jnp.float32)


def _dotf(a, b):
    return jnp.dot(a, b, preferred_element_type=jnp.float32)


def _tdot(a, b):
    return lax.dot_general(a.astype(_MM_DTYPE), b.astype(_MM_DTYPE),
                           _DIMS_T, preferred_element_type=jnp.float32)


def _tdotf(a, b):
    return lax.dot_general(a, b, _DIMS_T,
                           preferred_element_type=jnp.float32)


def _score(h, p):
    inv_norm = lax.rsqrt(jnp.sum(p * p))
    return jnp.tanh(jnp.sum(h * p, axis=1, keepdims=True) * inv_norm)


def _gcn(at, x, w, b, relu):
    deg = jnp.sum(at.astype(jnp.float32), axis=1, keepdims=True) + 2.0
    dinv = lax.rsqrt(deg)
    y = _dot(x, w) * dinv
    out = (_dot(at, y) + 2.0 * y) * dinv + b
    return jnp.maximum(out, 0.0) if relu else out


def _gcn_t(adj, x, w, b, relu):
    n = adj.shape[0]
    ones = jnp.ones((n, 1), jnp.float32)
    deg = _tdot(adj, ones) + 2.0
    dinv = lax.rsqrt(deg)
    y = _dot(x, w) * dinv
    out = (_tdot(adj, y) + 2.0 * y) * dinv + b
    return jnp.maximum(out, 0.0) if relu else out


def _pt_onehot(pr, n):
    kk = pr.shape[1]
    ri = lax.broadcasted_iota(jnp.int32, (n, kk), 0).astype(jnp.float32)
    return (pr[...] == ri).astype(jnp.float32)


def _offdiag(m):
    kk = m.shape[0]
    eye = (lax.broadcasted_iota(jnp.int32, (kk, kk), 0) ==
           lax.broadcasted_iota(jnp.int32, (kk, kk), 1)).astype(m.dtype)
    return m * (1.0 - eye)


def _conv_score_body(adj_ref, x_ref, w_ref, b_ref, p_ref, o_ref, s_ref):
    h = _gcn_t(adj_ref[...], x_ref[...], w_ref[...], b_ref[...], relu=True)
    o_ref[...] = h
    s_ref[...] = jnp.broadcast_to(_score(h, p_ref[...]), s_ref.shape)


def _pooled_adj(a, pt, transposed_in, exact_sel):
    d = _tdot if transposed_in else _dot
    c = d(a, pt) + pt
    tmp = d(a, c) + c
    sel = _tdotf if exact_sel else _tdot
    return _offdiag(sel(pt, tmp))


def _down_mid_body(a_ref, x_ref, s_ref, pr_ref, w_ref, b_ref, p_ref,
                   ao_ref, o_ref, so_ref, *, transposed_in, exact_sel):
    n = a_ref.shape[0]
    pt = _pt_onehot(pr_ref, n)
    atn = _pooled_adj(a_ref[...], pt, transposed_in, exact_sel)
    ao_ref[...] = atn.astype(ao_ref.dtype)
    sg = _tdotf(pt, s_ref[:, :1])
    xg = _tdotf(pt, x_ref[...]) * sg
    h = _gcn(atn, xg, w_ref[...], b_ref[...], relu=True)
    o_ref[...] = h
    so_ref[...] = jnp.broadcast_to(_score(h, p_ref[...]), so_ref.shape)


def _bottom_body(a_ref, x_ref, s_ref, pr_ref, wd_ref, bd_ref,
                 wu_ref, bu_ref, o_ref):
    n = a_ref.shape[0]
    pt = _pt_onehot(pr_ref, n)
    a = a_ref[...]
    c = _dotf(a, pt) + pt
    tmp = _dot(a, c) + c
    at3 = _offdiag(_tdotf(pt, tmp))
    sg = _tdotf(pt, s_ref[:, :1])
    xg = _tdotf(pt, x_ref[...]) * sg
    x3 = _gcn(at3, xg, wd_ref[...], bd_ref[...], relu=True)
    u = x_ref[...] + _dotf(pt, x3)
    o_ref[...] = _gcn(a, u, wu_ref[...], bu_ref[...], relu=True)


def _up_conv_body(pr_ref, xu_ref, res_ref, a_ref, w_ref, b_ref, o_ref, *,
                  relu, transposed_in):
    n = res_ref.shape[0]
    pt = _pt_onehot(pr_ref, n)
    u = res_ref[...] + _dotf(pt, xu_ref[...])
    conv = _gcn_t if transposed_in else _gcn
    o_ref[...] = conv(a_ref[...], u, w_ref[...], b_ref[...], relu)


def _full(shape):
    return pl.BlockSpec(shape, lambda i: (0,) * len(shape))


def _call(body, ins, outs):
    return pl.pallas_call(
        body,
        out_shape=outs,
        grid=(1,),
        in_specs=[_full(a.shape) for a in ins],
        out_specs=jax.tree.map(lambda s: _full(s.shape), outs),
    )(*ins)


def _conv_score(adj, x, w, b, p):
    n, co = adj.shape[0], w.shape[1]
    return _call(_conv_score_body, (adj, x, w, b, p),
                 (jax.ShapeDtypeStruct((n, co), jnp.float32),
                  jax.ShapeDtypeStruct((n, 128), jnp.float32)))


def _down_mid(a, x, s, pr, w, b, p, transposed_in, exact_sel, adj_dtype):
    kk, co = pr.shape[1], w.shape[1]
    body = functools.partial(_down_mid_body, transposed_in=transposed_in,
                             exact_sel=exact_sel)
    return _call(body, (a, x, s, pr, w, b, p),
                 (jax.ShapeDtypeStruct((kk, kk), adj_dtype),
                  jax.ShapeDtypeStruct((kk, co), jnp.float32),
                  jax.ShapeDtypeStruct((kk, 128), jnp.float32)))


def _bottom(a, x, s, pr, wd, bd, wu, bu):
    n, co = a.shape[0], wu.shape[1]
    return _call(_bottom_body, (a, x, s, pr, wd, bd, wu, bu),
                 jax.ShapeDtypeStruct((n, co), jnp.float32))


def _up_conv(pr, xu, res, a, w, b, relu, transposed_in):
    n, co = a.shape[0], w.shape[1]
    body = functools.partial(_up_conv_body, relu=relu,
                             transposed_in=transposed_in)
    return _call(body, (pr, xu, res, a, w, b),
                 jax.ShapeDtypeStruct((n, co), jnp.float32))


def _perm_row(s, kk):
    _, perm = lax.top_k(s, kk)
    return perm.astype(jnp.float32)[None, :]


def kernel(x, adj, down_w_0, down_w_1, down_w_2, down_w_3,
           down_b_0, down_b_1, down_b_2, down_b_3,
           pool_p_0, pool_p_1, pool_p_2,
           up_w_0, up_w_1, up_w_2,
           up_b_0, up_b_1, up_b_2):
    n = adj.shape[0]
    k1 = int(math.ceil(0.5 * n))
    k2 = int(math.ceil(0.5 * k1))
    k3 = int(math.ceil(0.5 * k2))
    adj_bf = adj.astype(jnp.bfloat16)

    x0, s0 = _conv_score(adj_bf, x, down_w_0, down_b_0, pool_p_0)
    pr1 = _perm_row(s0[:, 0], k1)
    at1, x1, s1 = _down_mid(adj_bf, x0, s0, pr1, down_w_1, down_b_1,
                            pool_p_1, transposed_in=True, exact_sel=False,
                            adj_dtype=jnp.bfloat16)
    pr2 = _perm_row(s1[:, 0], k2)
    at2, x2, s2 = _down_mid(at1, x1, s1, pr2, down_w_2, down_b_2,
                            pool_p_2, transposed_in=False, exact_sel=True,
                            adj_dtype=jnp.float32)
    pr3 = _perm_row(s2[:, 0], k3)
    u = _bottom(at2, x2, s2, pr3, down_w_3, down_b_3, up_w_0, up_b_0)
    u = _up_conv(pr2, u, x1, at1, up_w_1, up_b_1, relu=True,
                 transposed_in=False)
    return _up_conv(pr1, u, x0, adj_bf, up_w_2, up_b_2, relu=False,
                    transposed_in=True)


# speedup vs baseline: 4.6751x; 1.0926x over previous
import math

import functools

import jax
import jax.numpy as jnp
from jax import lax
from jax.experimental import pallas as pl

_MM_DTYPE = jnp.bfloat16
_DIMS_T = (((0,), (0,)), ((), ()))


def _dot(a, b):
    return jnp.dot(a.astype(_MM_DTYPE), b.astype(_MM_DTYPE),
                   preferred_element_type=jnp.float32)


def _dotf(a, b):
    return jnp.dot(a, b, preferred_element_type=jnp.float32)


def _tdot(a, b):
    return lax.dot_general(a.astype(_MM_DTYPE), b.astype(_MM_DTYPE),
                           _DIMS_T, preferred_element_type=jnp.float32)


def _tdotf(a, b):
    return lax.dot_general(a, b, _DIMS_T,
                           preferred_element_type=jnp.float32)


def _score(h, p):
    inv_norm = lax.rsqrt(jnp.sum(p * p))
    return jnp.tanh(jnp.sum(h * p, axis=1, keepdims=True) * inv_norm)


def _gcn(at, x, w, b, relu):
    deg = jnp.sum(at.astype(jnp.float32), axis=1, keepdims=True) + 2.0
    dinv = lax.rsqrt(deg)
    y = _dot(x, w) * dinv
    out = (_dot(at, y) + 2.0 * y) * dinv + b
    return jnp.maximum(out, 0.0) if relu else out


def _gcn_t(adj, x, w, b, relu):
    n = adj.shape[0]
    ones = jnp.ones((n, 1), jnp.float32)
    deg = _tdot(adj, ones) + 2.0
    dinv = lax.rsqrt(deg)
    y = _dot(x, w) * dinv
    out = (_tdot(adj, y) + 2.0 * y) * dinv + b
    return jnp.maximum(out, 0.0) if relu else out


def _pt_onehot(pr, n):
    kk = pr.shape[1]
    ri = lax.broadcasted_iota(jnp.int32, (n, kk), 0).astype(jnp.float32)
    return (pr[...] == ri).astype(jnp.float32)


def _offdiag(m):
    kk = m.shape[0]
    eye = (lax.broadcasted_iota(jnp.int32, (kk, kk), 0) ==
           lax.broadcasted_iota(jnp.int32, (kk, kk), 1)).astype(m.dtype)
    return m * (1.0 - eye)


def _conv_score_body(adj_ref, x_ref, w_ref, b_ref, p_ref, o_ref, s_ref):
    h = _gcn_t(adj_ref[...], x_ref[...], w_ref[...], b_ref[...], relu=True)
    o_ref[...] = h
    s_ref[...] = jnp.broadcast_to(_score(h, p_ref[...]), s_ref.shape)


def _pooled_adj(a, pt, transposed_in, exact_sel):
    d = _tdot if transposed_in else _dot
    c = d(a, pt) + pt
    tmp = d(a, c) + c
    sel = _tdotf if exact_sel else _tdot
    return _offdiag(sel(pt, tmp))


def _down_mid_body(a_ref, x_ref, s_ref, pr_ref, w_ref, b_ref, p_ref,
                   ao_ref, o_ref, so_ref, *, transposed_in, exact_sel):
    n = a_ref.shape[0]
    pt = _pt_onehot(pr_ref, n)
    atn = _pooled_adj(a_ref[...], pt, transposed_in, exact_sel)
    ao_ref[...] = atn.astype(ao_ref.dtype)
    sg = _tdotf(pt, s_ref[:, :1])
    xg = _tdotf(pt, x_ref[...]) * sg
    h = _gcn(atn, xg, w_ref[...], b_ref[...], relu=True)
    o_ref[...] = h
    so_ref[...] = jnp.broadcast_to(_score(h, p_ref[...]), so_ref.shape)


def _finale_body(a2_ref, x2_ref, s2_ref, pr3_ref, at1_ref, x1_ref, pr2_ref,
                 adj_ref, x0_ref, pr1_ref, wd_ref, bd_ref, wu0_ref, bu0_ref,
                 wu1_ref, bu1_ref, wu2_ref, bu2_ref, o_ref):
    n2 = a2_ref.shape[0]
    pt3 = _pt_onehot(pr3_ref, n2)
    a2 = a2_ref[...]
    c = _dotf(a2, pt3) + pt3
    tmp = _dot(a2, c) + c
    at3 = _offdiag(_tdotf(pt3, tmp))
    sg = _tdotf(pt3, s2_ref[:, :1])
    xg = _tdotf(pt3, x2_ref[...]) * sg
    x3 = _gcn(at3, xg, wd_ref[...], bd_ref[...], relu=True)
    u2 = x2_ref[...] + _dotf(pt3, x3)
    h2 = _gcn(a2, u2, wu0_ref[...], bu0_ref[...], relu=True)
    pt2 = _pt_onehot(pr2_ref, at1_ref.shape[0])
    u1 = x1_ref[...] + _dotf(pt2, h2)
    h1 = _gcn(at1_ref[...], u1, wu1_ref[...], bu1_ref[...], relu=True)
    pt1 = _pt_onehot(pr1_ref, adj_ref.shape[0])
    u0 = x0_ref[...] + _dotf(pt1, h1)
    o_ref[...] = _gcn_t(adj_ref[...], u0, wu2_ref[...], bu2_ref[...],
                        relu=False)


def _full(shape):
    return pl.BlockSpec(shape, lambda i: (0,) * len(shape))


def _call(body, ins, outs):
    return pl.pallas_call(
        body,
        out_shape=outs,
        grid=(1,),
        in_specs=[_full(a.shape) for a in ins],
        out_specs=jax.tree.map(lambda s: _full(s.shape), outs),
    )(*ins)


def _conv_score(adj, x, w, b, p):
    n, co = adj.shape[0], w.shape[1]
    return _call(_conv_score_body, (adj, x, w, b, p),
                 (jax.ShapeDtypeStruct((n, co), jnp.float32),
                  jax.ShapeDtypeStruct((n, 128), jnp.float32)))


def _down_mid(a, x, s, pr, w, b, p, transposed_in, exact_sel, adj_dtype):
    kk, co = pr.shape[1], w.shape[1]
    body = functools.partial(_down_mid_body, transposed_in=transposed_in,
                             exact_sel=exact_sel)
    return _call(body, (a, x, s, pr, w, b, p),
                 (jax.ShapeDtypeStruct((kk, kk), adj_dtype),
                  jax.ShapeDtypeStruct((kk, co), jnp.float32),
                  jax.ShapeDtypeStruct((kk, 128), jnp.float32)))


def _finale(a2, x2, s2, pr3, at1, x1, pr2, adj, x0, pr1,
            wd, bd, wu0, bu0, wu1, bu1, wu2, bu2):
    n, co = adj.shape[0], wu2.shape[1]
    return _call(_finale_body,
                 (a2, x2, s2, pr3, at1, x1, pr2, adj, x0, pr1,
                  wd, bd, wu0, bu0, wu1, bu1, wu2, bu2),
                 jax.ShapeDtypeStruct((n, co), jnp.float32))


def _perm_row(s, kk):
    _, perm = lax.top_k(s, kk)
    return perm.astype(jnp.float32)[None, :]


def kernel(x, adj, down_w_0, down_w_1, down_w_2, down_w_3,
           down_b_0, down_b_1, down_b_2, down_b_3,
           pool_p_0, pool_p_1, pool_p_2,
           up_w_0, up_w_1, up_w_2,
           up_b_0, up_b_1, up_b_2):
    n = adj.shape[0]
    k1 = int(math.ceil(0.5 * n))
    k2 = int(math.ceil(0.5 * k1))
    k3 = int(math.ceil(0.5 * k2))
    adj_bf = adj.astype(jnp.bfloat16)

    x0, s0 = _conv_score(adj_bf, x, down_w_0, down_b_0, pool_p_0)
    pr1 = _perm_row(s0[:, 0], k1)
    at1, x1, s1 = _down_mid(adj_bf, x0, s0, pr1, down_w_1, down_b_1,
                            pool_p_1, transposed_in=True, exact_sel=False,
                            adj_dtype=jnp.bfloat16)
    pr2 = _perm_row(s1[:, 0], k2)
    at2, x2, s2 = _down_mid(at1, x1, s1, pr2, down_w_2, down_b_2,
                            pool_p_2, transposed_in=False, exact_sel=True,
                            adj_dtype=jnp.float32)
    pr3 = _perm_row(s2[:, 0], k3)
    return _finale(at2, x2, s2, pr3, at1, x1, pr2, adj_bf, x0, pr1,
                   down_w_3, down_b_3, up_w_0, up_b_0, up_w_1, up_b_1,
                   up_w_2, up_b_2)


# speedup vs baseline: 4.8438x; 1.0361x over previous
import math

import functools

import jax
import jax.numpy as jnp
from jax import lax
from jax.experimental import pallas as pl

_MM_DTYPE = jnp.bfloat16
_DIMS_T = (((0,), (0,)), ((), ()))


def _dot(a, b):
    return jnp.dot(a.astype(_MM_DTYPE), b.astype(_MM_DTYPE),
                   preferred_element_type=jnp.float32)


def _dotf(a, b):
    return jnp.dot(a, b, preferred_element_type=jnp.float32)


def _tdot(a, b):
    return lax.dot_general(a.astype(_MM_DTYPE), b.astype(_MM_DTYPE),
                           _DIMS_T, preferred_element_type=jnp.float32)


def _tdotf(a, b):
    return lax.dot_general(a, b, _DIMS_T,
                           preferred_element_type=jnp.float32)


def _score(h, p):
    inv_norm = lax.rsqrt(jnp.sum(p * p))
    return jnp.tanh(jnp.sum(h * p, axis=1, keepdims=True) * inv_norm)


def _gcn(at, x, w, b, relu):
    deg = jnp.sum(at.astype(jnp.float32), axis=1, keepdims=True) + 2.0
    dinv = lax.rsqrt(deg)
    y = _dot(x, w) * dinv
    out = (_dot(at, y) + 2.0 * y) * dinv + b
    return jnp.maximum(out, 0.0) if relu else out


def _gcn_t(adj, x, w, b, relu):
    n = adj.shape[0]
    ones = jnp.ones((n, 1), jnp.float32)
    deg = _tdot(adj, ones) + 2.0
    dinv = lax.rsqrt(deg)
    y = _dot(x, w) * dinv
    out = (_tdot(adj, y) + 2.0 * y) * dinv + b
    return jnp.maximum(out, 0.0) if relu else out


def _pt_onehot(pr, n):
    kk = pr.shape[1]
    ri = lax.broadcasted_iota(jnp.int32, (n, kk), 0).astype(jnp.float32)
    return (pr[...] == ri).astype(jnp.float32)


def _offdiag(m):
    kk = m.shape[0]
    eye = (lax.broadcasted_iota(jnp.int32, (kk, kk), 0) ==
           lax.broadcasted_iota(jnp.int32, (kk, kk), 1)).astype(m.dtype)
    return m * (1.0 - eye)


def _conv_score_body(adj_ref, x_ref, w_ref, b_ref, p_ref, o_ref, s_ref):
    h = _gcn_t(adj_ref[...], x_ref[...], w_ref[...], b_ref[...], relu=True)
    o_ref[...] = h
    s_ref[...] = jnp.broadcast_to(_score(h, p_ref[...]), s_ref.shape)


def _pooled_adj(a, pt, transposed_in, exact_sel):
    d = _tdot if transposed_in else _dot
    c = d(a, pt) + pt
    tmp = d(a, c) + c
    sel = _tdotf if exact_sel else _tdot
    return _offdiag(sel(pt, tmp))


def _down_mid_body(a_ref, x_ref, s_ref, pr_ref, w_ref, b_ref, p_ref,
                   ao_ref, o_ref, so_ref, *, transposed_in, exact_sel):
    n = a_ref.shape[0]
    pt = _pt_onehot(pr_ref, n)
    atn = _pooled_adj(a_ref[...], pt, transposed_in, exact_sel)
    ao_ref[...] = atn.astype(ao_ref.dtype)
    sg = _tdotf(pt, s_ref[:, :1])
    xg = _tdotf(pt, x_ref[...]) * sg
    h = _gcn(atn, xg, w_ref[...], b_ref[...], relu=True)
    o_ref[...] = h
    so_ref[...] = jnp.broadcast_to(_score(h, p_ref[...]), so_ref.shape)


def _finale_body(a2_ref, x2_ref, s2_ref, pr3_ref, at1_ref, x1_ref, pr2_ref,
                 adj_ref, x0_ref, pr1_ref, wd_ref, bd_ref, wu0_ref, bu0_ref,
                 wu1_ref, bu1_ref, wu2_ref, bu2_ref, o_ref):
    n2 = a2_ref.shape[0]
    pt3 = _pt_onehot(pr3_ref, n2)
    a2 = a2_ref[...]
    c = _dotf(a2, pt3) + pt3
    tmp = _dot(a2, c) + c
    at3 = _offdiag(_tdotf(pt3, tmp))
    sg = _tdotf(pt3, s2_ref[:, :1])
    xg = _tdotf(pt3, x2_ref[...]) * sg
    x3 = _gcn(at3, xg, wd_ref[...], bd_ref[...], relu=True)
    u2 = x2_ref[...] + _dotf(pt3, x3)
    h2 = _gcn(a2, u2, wu0_ref[...], bu0_ref[...], relu=True)
    pt2 = _pt_onehot(pr2_ref, at1_ref.shape[0])
    u1 = x1_ref[...] + _dotf(pt2, h2)
    h1 = _gcn(at1_ref[...], u1, wu1_ref[...], bu1_ref[...], relu=True)
    pt1 = _pt_onehot(pr1_ref, adj_ref.shape[0])
    u0 = x0_ref[...] + _dotf(pt1, h1)
    o_ref[...] = _gcn_t(adj_ref[...], u0, wu2_ref[...], bu2_ref[...],
                        relu=False)


def _full(shape):
    return pl.BlockSpec(shape, lambda i: (0,) * len(shape))


def _call(body, ins, outs):
    return pl.pallas_call(
        body,
        out_shape=outs,
        grid=(1,),
        in_specs=[_full(a.shape) for a in ins],
        out_specs=jax.tree.map(lambda s: _full(s.shape), outs),
    )(*ins)


def _conv_score(adj, x, w, b, p):
    n, co = adj.shape[0], w.shape[1]
    return _call(_conv_score_body, (adj, x, w, b, p),
                 (jax.ShapeDtypeStruct((n, co), jnp.float32),
                  jax.ShapeDtypeStruct((n, 1), jnp.float32)))


def _down_mid(a, x, s, pr, w, b, p, transposed_in, exact_sel, adj_dtype):
    kk, co = pr.shape[1], w.shape[1]
    body = functools.partial(_down_mid_body, transposed_in=transposed_in,
                             exact_sel=exact_sel)
    return _call(body, (a, x, s, pr, w, b, p),
                 (jax.ShapeDtypeStruct((kk, kk), adj_dtype),
                  jax.ShapeDtypeStruct((kk, co), jnp.float32),
                  jax.ShapeDtypeStruct((kk, 1), jnp.float32)))


def _finale(a2, x2, s2, pr3, at1, x1, pr2, adj, x0, pr1,
            wd, bd, wu0, bu0, wu1, bu1, wu2, bu2):
    n, co = adj.shape[0], wu2.shape[1]
    return _call(_finale_body,
                 (a2, x2, s2, pr3, at1, x1, pr2, adj, x0, pr1,
                  wd, bd, wu0, bu0, wu1, bu1, wu2, bu2),
                 jax.ShapeDtypeStruct((n, co), jnp.float32))


def _perm_row(s, kk):
    _, perm = lax.top_k(s, kk)
    return perm.astype(jnp.float32)[None, :]


def kernel(x, adj, down_w_0, down_w_1, down_w_2, down_w_3,
           down_b_0, down_b_1, down_b_2, down_b_3,
           pool_p_0, pool_p_1, pool_p_2,
           up_w_0, up_w_1, up_w_2,
           up_b_0, up_b_1, up_b_2):
    n = adj.shape[0]
    k1 = int(math.ceil(0.5 * n))
    k2 = int(math.ceil(0.5 * k1))
    k3 = int(math.ceil(0.5 * k2))
    x0, s0 = _conv_score(adj, x, down_w_0, down_b_0, pool_p_0)
    pr1 = _perm_row(s0[:, 0], k1)
    at1, x1, s1 = _down_mid(adj, x0, s0, pr1, down_w_1, down_b_1,
                            pool_p_1, transposed_in=True, exact_sel=False,
                            adj_dtype=jnp.bfloat16)
    pr2 = _perm_row(s1[:, 0], k2)
    at2, x2, s2 = _down_mid(at1, x1, s1, pr2, down_w_2, down_b_2,
                            pool_p_2, transposed_in=False, exact_sel=True,
                            adj_dtype=jnp.float32)
    pr3 = _perm_row(s2[:, 0], k3)
    return _finale(at2, x2, s2, pr3, at1, x1, pr2, adj, x0, pr1,
                   down_w_3, down_b_3, up_w_0, up_b_0, up_w_1, up_b_1,
                   up_w_2, up_b_2)


# speedup vs baseline: 6.6154x; 1.3658x over previous
import functools
import math

import jax
import jax.numpy as jnp
from jax import lax
from jax.experimental import pallas as pl

_MM_DTYPE = jnp.bfloat16
_DIMS_T = (((0,), (0,)), ((), ()))


def _dot(a, b):
    return jnp.dot(a.astype(_MM_DTYPE), b.astype(_MM_DTYPE),
                   preferred_element_type=jnp.float32)


def _dotf(a, b):
    return jnp.dot(a, b, preferred_element_type=jnp.float32)


def _tdot(a, b):
    return lax.dot_general(a.astype(_MM_DTYPE), b.astype(_MM_DTYPE),
                           _DIMS_T, preferred_element_type=jnp.float32)


def _tdotf(a, b):
    return lax.dot_general(a, b, _DIMS_T,
                           preferred_element_type=jnp.float32)


def _score(h, p):
    inv_norm = lax.rsqrt(jnp.sum(p * p))
    return jnp.tanh(jnp.sum(h * p, axis=1, keepdims=True) * inv_norm)


def _rank(s):
    n = s.shape[0]
    eye = (lax.broadcasted_iota(jnp.int32, (n, n), 0) ==
           lax.broadcasted_iota(jnp.int32, (n, n), 1)).astype(jnp.float32)
    srow = _tdotf(s, eye)
    blk = min(128, n)
    ri = lax.broadcasted_iota(jnp.int32, (n, blk), 0)
    cj = lax.broadcasted_iota(jnp.int32, (n, blk), 1)
    cnt = jnp.zeros((n, 1), jnp.float32)
    for j0 in range(0, n, blk):
        sc = srow[:, j0:j0 + blk]
        beats = (sc > s) | ((sc == s) & ((j0 + cj) < ri))
        cnt = cnt + jnp.sum(beats.astype(jnp.float32), axis=1, keepdims=True)
    return cnt


def _rank_pt(rank, kk):
    n = rank.shape[0]
    ji = lax.broadcasted_iota(jnp.int32, (n, kk), 1).astype(jnp.float32)
    return (rank == ji).astype(jnp.float32)


def _gcn(at, x, w, b, relu):
    deg = jnp.sum(at.astype(jnp.float32), axis=1, keepdims=True) + 2.0
    dinv = lax.rsqrt(deg)
    y = _dot(x, w) * dinv
    out = (_dot(at, y) + 2.0 * y) * dinv + b
    return jnp.maximum(out, 0.0) if relu else out


def _gcn_t(adj, x, w, b, relu):
    n = adj.shape[0]
    ones = jnp.ones((n, 1), jnp.float32)
    deg = _tdot(adj, ones) + 2.0
    dinv = lax.rsqrt(deg)
    y = _dot(x, w) * dinv
    out = (_tdot(adj, y) + 2.0 * y) * dinv + b
    return jnp.maximum(out, 0.0) if relu else out


def _offdiag(m):
    kk = m.shape[0]
    eye = (lax.broadcasted_iota(jnp.int32, (kk, kk), 0) ==
           lax.broadcasted_iota(jnp.int32, (kk, kk), 1)).astype(m.dtype)
    return m * (1.0 - eye)


def _pooled_adj(a, pt, transposed_in, exact_sel):
    if exact_sel:
        d0 = _tdotf if transposed_in else _dotf
        c = d0(a, pt) + pt
        tmp = _dot(a, c) + c
        return _offdiag(_tdotf(pt, tmp))
    d0 = _tdot if transposed_in else _dot
    c = d0(a, pt) + pt
    tmp = d0(a, c) + c
    return _offdiag(_tdot(pt, tmp))


def _conv_score_body(adj_ref, x_ref, w_ref, b_ref, p_ref,
                     o_ref, s_ref, r_ref):
    h = _gcn_t(adj_ref[...], x_ref[...], w_ref[...], b_ref[...], relu=True)
    o_ref[...] = h
    s = _score(h, p_ref[...])
    s_ref[...] = s
    r_ref[...] = _rank(s)


def _down_mid_body(a_ref, x_ref, s_ref, rk_ref, w_ref, b_ref, p_ref,
                   ao_ref, o_ref, so_ref, ro_ref, *, kk, transposed_in,
                   exact_sel):
    pt = _rank_pt(rk_ref[...], kk)
    atn = _pooled_adj(a_ref[...], pt, transposed_in, exact_sel)
    ao_ref[...] = atn.astype(ao_ref.dtype)
    xg = _tdotf(pt, x_ref[...]) * _tdotf(pt, s_ref[...])
    h = _gcn(atn, xg, w_ref[...], b_ref[...], relu=True)
    o_ref[...] = h
    s = _score(h, p_ref[...])
    so_ref[...] = s
    ro_ref[...] = _rank(s)


def _finale_body(a2_ref, x2_ref, s2_ref, rk3_ref, at1_ref, x1_ref, rk2_ref,
                 adj_ref, x0_ref, rk1_ref, wd_ref, bd_ref, wu0_ref, bu0_ref,
                 wu1_ref, bu1_ref, wu2_ref, bu2_ref, o_ref, *, ks):
    k1, k2, k3 = ks
    pt3 = _rank_pt(rk3_ref[...], k3)
    a2 = a2_ref[...]
    at3 = _pooled_adj(a2, pt3, transposed_in=False, exact_sel=True)
    xg = _tdotf(pt3, x2_ref[...]) * _tdotf(pt3, s2_ref[...])
    x3 = _gcn(at3, xg, wd_ref[...], bd_ref[...], relu=True)
    h2 = _gcn(a2, x2_ref[...] + _dotf(pt3, x3), wu0_ref[...], bu0_ref[...],
              relu=True)
    pt2 = _rank_pt(rk2_ref[...], k2)
    h1 = _gcn(at1_ref[...], x1_ref[...] + _dotf(pt2, h2), wu1_ref[...],
              bu1_ref[...], relu=True)
    pt1 = _rank_pt(rk1_ref[...], k1)
    o_ref[...] = _gcn_t(adj_ref[...], x0_ref[...] + _dotf(pt1, h1),
                        wu2_ref[...], bu2_ref[...], relu=False)


def _full(shape):
    return pl.BlockSpec(shape, lambda i: (0,) * len(shape))


def _call(body, ins, outs):
    return pl.pallas_call(
        body,
        out_shape=outs,
        grid=(1,),
        in_specs=[_full(a.shape) for a in ins],
        out_specs=jax.tree.map(lambda s: _full(s.shape), outs),
    )(*ins)


def kernel(x, adj, down_w_0, down_w_1, down_w_2, down_w_3,
           down_b_0, down_b_1, down_b_2, down_b_3,
           pool_p_0, pool_p_1, pool_p_2,
           up_w_0, up_w_1, up_w_2,
           up_b_0, up_b_1, up_b_2):
    n, co = adj.shape[0], up_w_2.shape[1]
    k1 = int(math.ceil(0.5 * n))
    k2 = int(math.ceil(0.5 * k1))
    k3 = int(math.ceil(0.5 * k2))
    f32 = jnp.float32
    cdim = down_w_0.shape[1]
    v = lambda m: jax.ShapeDtypeStruct((m, 1), f32)

    x0, s0, rk1 = _call(
        _conv_score_body, (adj, x, down_w_0, down_b_0, pool_p_0),
        (jax.ShapeDtypeStruct((n, cdim), f32), v(n), v(n)))
    at1, x1, s1, rk2 = _call(
        functools.partial(_down_mid_body, kk=k1, transposed_in=True,
                          exact_sel=False),
        (adj, x0, s0, rk1, down_w_1, down_b_1, pool_p_1),
        (jax.ShapeDtypeStruct((k1, k1), jnp.bfloat16),
         jax.ShapeDtypeStruct((k1, cdim), f32), v(k1), v(k1)))
    at2, x2, s2, rk3 = _call(
        functools.partial(_down_mid_body, kk=k2, transposed_in=False,
                          exact_sel=True),
        (at1, x1, s1, rk2, down_w_2, down_b_2, pool_p_2),
        (jax.ShapeDtypeStruct((k2, k2), f32),
         jax.ShapeDtypeStruct((k2, cdim), f32), v(k2), v(k2)))
    return _call(
        functools.partial(_finale_body, ks=(k1, k2, k3)),
        (at2, x2, s2, rk3, at1, x1, rk2, adj, x0, rk1,
         down_w_3, down_b_3, up_w_0, up_b_0, up_w_1, up_b_1,
         up_w_2, up_b_2),
        jax.ShapeDtypeStruct((n, co), f32))


# speedup vs baseline: 9.9124x; 1.4984x over previous
import functools
import math

import jax
import jax.numpy as jnp
from jax import lax
from jax.experimental import pallas as pl

_MM_DTYPE = jnp.bfloat16
_DIMS_T = (((0,), (0,)), ((), ()))


def _dot(a, b):
    return jnp.dot(a.astype(_MM_DTYPE), b.astype(_MM_DTYPE),
                   preferred_element_type=jnp.float32)


def _dotf(a, b):
    return jnp.dot(a, b, preferred_element_type=jnp.float32)


def _tdot(a, b):
    return lax.dot_general(a.astype(_MM_DTYPE), b.astype(_MM_DTYPE),
                           _DIMS_T, preferred_element_type=jnp.float32)


def _tdotf(a, b):
    return lax.dot_general(a, b, _DIMS_T,
                           preferred_element_type=jnp.float32)


def _score(h, p):
    inv_norm = lax.rsqrt(jnp.sum(p * p))
    return jnp.tanh(jnp.sum(h * p, axis=1, keepdims=True) * inv_norm)


def _rank(s):
    n = s.shape[0]
    eye = (lax.broadcasted_iota(jnp.int32, (n, n), 0) ==
           lax.broadcasted_iota(jnp.int32, (n, n), 1)).astype(jnp.float32)
    srow = _tdotf(s, eye)
    blk = min(128, n)
    ri = lax.broadcasted_iota(jnp.int32, (n, blk), 0)
    cj = lax.broadcasted_iota(jnp.int32, (n, blk), 1)
    cnt = jnp.zeros((n, 1), jnp.float32)
    for j0 in range(0, n, blk):
        sc = srow[:, j0:j0 + blk]
        beats = (sc > s) | ((sc == s) & ((j0 + cj) < ri))
        cnt = cnt + jnp.sum(beats.astype(jnp.float32), axis=1, keepdims=True)
    return cnt


def _rank_pt(rank, kk):
    n = rank.shape[0]
    ji = lax.broadcasted_iota(jnp.int32, (n, kk), 1).astype(jnp.float32)
    return (rank == ji).astype(jnp.float32)


def _gcn(at, x, w, b, relu):
    deg = jnp.sum(at.astype(jnp.float32), axis=1, keepdims=True) + 2.0
    dinv = lax.rsqrt(deg)
    y = _dot(x, w) * dinv
    out = (_dot(at, y) + 2.0 * y) * dinv + b
    return jnp.maximum(out, 0.0) if relu else out


def _gcn_t(adj, x, w, b, relu):
    n = adj.shape[0]
    ones = jnp.ones((n, 1), jnp.float32)
    deg = _tdot(adj, ones) + 2.0
    dinv = lax.rsqrt(deg)
    y = _dot(x, w) * dinv
    out = (_tdot(adj, y) + 2.0 * y) * dinv + b
    return jnp.maximum(out, 0.0) if relu else out


def _offdiag(m):
    kk = m.shape[0]
    eye = (lax.broadcasted_iota(jnp.int32, (kk, kk), 0) ==
           lax.broadcasted_iota(jnp.int32, (kk, kk), 1)).astype(m.dtype)
    return m * (1.0 - eye)


def _pooled_adj(a, pt, transposed_in, exact_sel):
    if exact_sel:
        d0 = _tdotf if transposed_in else _dotf
        c = d0(a, pt) + pt
        tmp = _dot(a, c) + c
        return _offdiag(_tdotf(pt, tmp))
    d0 = _tdot if transposed_in else _dot
    c = d0(a, pt) + pt
    tmp = d0(a, c) + c
    return _offdiag(_tdot(pt, tmp))


def _conv_score_body(adj_ref, x_ref, w_ref, b_ref, p_ref,
                     o_ref, s_ref, r_ref):
    h = _gcn_t(adj_ref[...], x_ref[...], w_ref[...], b_ref[...], relu=True)
    o_ref[...] = h
    s = _score(h, p_ref[...])
    s_ref[...] = s
    r_ref[...] = _rank(s)


def _down_mid_body(a_ref, x_ref, s_ref, rk_ref, w_ref, b_ref, p_ref,
                   ao_ref, o_ref, so_ref, ro_ref, *, kk, transposed_in,
                   exact_sel):
    pt = _rank_pt(rk_ref[...], kk)
    atn = _pooled_adj(a_ref[...], pt, transposed_in, exact_sel)
    ao_ref[...] = atn.astype(ao_ref.dtype)
    xg = _tdotf(pt, x_ref[...]) * _tdotf(pt, s_ref[...])
    h = _gcn(atn, xg, w_ref[...], b_ref[...], relu=True)
    o_ref[...] = h
    s = _score(h, p_ref[...])
    so_ref[...] = s
    ro_ref[...] = _rank(s)


def _finale_body(a2_ref, x2_ref, s2_ref, rk3_ref, at1_ref, x1_ref, rk2_ref,
                 adj_ref, x0_ref, rk1_ref, wd_ref, bd_ref, wu0_ref, bu0_ref,
                 wu1_ref, bu1_ref, wu2_ref, bu2_ref, o_ref, *, ks):
    k1, k2, k3 = ks
    pt3 = _rank_pt(rk3_ref[...], k3)
    a2 = a2_ref[...]
    at3 = _pooled_adj(a2, pt3, transposed_in=False, exact_sel=True)
    xg = _tdotf(pt3, x2_ref[...]) * _tdotf(pt3, s2_ref[...])
    x3 = _gcn(at3, xg, wd_ref[...], bd_ref[...], relu=True)
    h2 = _gcn(a2, x2_ref[...] + _dotf(pt3, x3), wu0_ref[...], bu0_ref[...],
              relu=True)
    pt2 = _rank_pt(rk2_ref[...], k2)
    h1 = _gcn(at1_ref[...], x1_ref[...] + _dotf(pt2, h2), wu1_ref[...],
              bu1_ref[...], relu=True)
    pt1 = _rank_pt(rk1_ref[...], k1)
    o_ref[...] = _gcn_t(adj_ref[...], x0_ref[...] + _dotf(pt1, h1),
                        wu2_ref[...], bu2_ref[...], relu=False)


def _full(shape):
    return pl.BlockSpec(shape, lambda i: (0,) * len(shape))


def _call(body, ins, outs):
    return pl.pallas_call(
        body,
        out_shape=outs,
        grid=(1,),
        in_specs=[_full(a.shape) for a in ins],
        out_specs=jax.tree.map(lambda s: _full(s.shape), outs),
    )(*ins)


def _unet_body(adj_ref, x_ref, w0_ref, w1_ref, w2_ref, w3_ref,
               b0_ref, b1_ref, b2_ref, b3_ref, p0_ref, p1_ref, p2_ref,
               u0_ref, u1_ref, u2_ref, c0_ref, c1_ref, c2_ref,
               o_ref, *, ks):
    k1, k2, k3 = ks
    adj = adj_ref[...]
    h0 = _gcn_t(adj, x_ref[...], w0_ref[...], b0_ref[...], relu=True)
    s0 = _score(h0, p0_ref[...])
    pt1 = _rank_pt(_rank(s0), k1)
    at1 = _pooled_adj(adj, pt1, transposed_in=True, exact_sel=False)
    h1 = _gcn(at1, _tdotf(pt1, h0) * _tdotf(pt1, s0),
              w1_ref[...], b1_ref[...], relu=True)
    s1 = _score(h1, p1_ref[...])
    pt2 = _rank_pt(_rank(s1), k2)
    at2 = _pooled_adj(at1, pt2, transposed_in=False, exact_sel=True)
    h2 = _gcn(at2, _tdotf(pt2, h1) * _tdotf(pt2, s1),
              w2_ref[...], b2_ref[...], relu=True)
    s2 = _score(h2, p2_ref[...])
    pt3 = _rank_pt(_rank(s2), k3)
    at3 = _pooled_adj(at2, pt3, transposed_in=False, exact_sel=True)
    x3 = _gcn(at3, _tdotf(pt3, h2) * _tdotf(pt3, s2),
              w3_ref[...], b3_ref[...], relu=True)
    g2 = _gcn(at2, h2 + _dotf(pt3, x3), u0_ref[...], c0_ref[...], relu=True)
    g1 = _gcn(at1, h1 + _dotf(pt2, g2), u1_ref[...], c1_ref[...], relu=True)
    o_ref[...] = _gcn_t(adj, h0 + _dotf(pt1, g1), u2_ref[...], c2_ref[...],
                        relu=False)


def kernel(x, adj, down_w_0, down_w_1, down_w_2, down_w_3,
           down_b_0, down_b_1, down_b_2, down_b_3,
           pool_p_0, pool_p_1, pool_p_2,
           up_w_0, up_w_1, up_w_2,
           up_b_0, up_b_1, up_b_2):
    n, co = adj.shape[0], up_w_2.shape[1]
    k1 = int(math.ceil(0.5 * n))
    k2 = int(math.ceil(0.5 * k1))
    k3 = int(math.ceil(0.5 * k2))
    ins = (adj, x, down_w_0, down_w_1, down_w_2, down_w_3,
           down_b_0, down_b_1, down_b_2, down_b_3,
           pool_p_0, pool_p_1, pool_p_2,
           up_w_0, up_w_1, up_w_2, up_b_0, up_b_1, up_b_2)
    return _call(functools.partial(_unet_body, ks=(k1, k2, k3)), ins,
                 jax.ShapeDtypeStruct((n, co), jnp.float32))


# speedup vs baseline: 9.9327x; 1.0020x over previous
import functools
import math

import jax
import jax.numpy as jnp
from jax import lax
from jax.experimental import pallas as pl

_MM_DTYPE = jnp.bfloat16
_DIMS_T = (((0,), (0,)), ((), ()))


def _dot(a, b):
    return jnp.dot(a.astype(_MM_DTYPE), b.astype(_MM_DTYPE),
                   preferred_element_type=jnp.float32)


def _dotf(a, b):
    return jnp.dot(a, b, preferred_element_type=jnp.float32)


def _tdot(a, b):
    return lax.dot_general(a.astype(_MM_DTYPE), b.astype(_MM_DTYPE),
                           _DIMS_T, preferred_element_type=jnp.float32)


def _tdotf(a, b):
    return lax.dot_general(a, b, _DIMS_T,
                           preferred_element_type=jnp.float32)


def _score(h, p):
    inv_norm = lax.rsqrt(jnp.sum(p * p))
    return jnp.tanh(jnp.sum(h * p, axis=1, keepdims=True) * inv_norm)


def _rank(s):
    n = s.shape[0]
    eye = (lax.broadcasted_iota(jnp.int32, (n, n), 0) ==
           lax.broadcasted_iota(jnp.int32, (n, n), 1)).astype(jnp.float32)
    srow = _tdotf(s, eye)
    blk = min(128, n)
    ri = lax.broadcasted_iota(jnp.int32, (n, blk), 0)
    cj = lax.broadcasted_iota(jnp.int32, (n, blk), 1)
    cnt = jnp.zeros((n, 1), jnp.float32)
    for j0 in range(0, n, blk):
        sc = srow[:, j0:j0 + blk]
        beats = (sc > s) | ((sc == s) & ((j0 + cj) < ri))
        cnt = cnt + jnp.sum(beats.astype(jnp.float32), axis=1, keepdims=True)
    return cnt


def _rank_pt(rank, kk):
    n = rank.shape[0]
    ji = lax.broadcasted_iota(jnp.int32, (n, kk), 1).astype(jnp.float32)
    return (rank == ji).astype(jnp.float32)


def _gcn(at, x, w, b, relu):
    deg = jnp.sum(at.astype(jnp.float32), axis=1, keepdims=True) + 2.0
    dinv = lax.rsqrt(deg)
    y = _dot(x, w) * dinv
    out = (_dot(at, y) + 2.0 * y) * dinv + b
    return jnp.maximum(out, 0.0) if relu else out


def _gcn_t(adj, x, w, b, relu):
    n = adj.shape[0]
    ones = jnp.ones((n, 1), jnp.float32)
    deg = _tdot(adj, ones) + 2.0
    dinv = lax.rsqrt(deg)
    y = _dot(x, w) * dinv
    out = (_tdot(adj, y) + 2.0 * y) * dinv + b
    return jnp.maximum(out, 0.0) if relu else out


def _offdiag(m):
    kk = m.shape[0]
    eye = (lax.broadcasted_iota(jnp.int32, (kk, kk), 0) ==
           lax.broadcasted_iota(jnp.int32, (kk, kk), 1)).astype(m.dtype)
    return m * (1.0 - eye)


def _pooled_adj(a, pt, transposed_in, exact_sel):
    if exact_sel:
        d0 = _tdotf if transposed_in else _dotf
        c = d0(a, pt) + pt
        tmp = _dot(a, c) + c
        return _offdiag(_tdotf(pt, tmp))
    d0 = _tdot if transposed_in else _dot
    c = d0(a, pt) + pt
    tmp = d0(a, c) + c
    return _offdiag(_tdot(pt, tmp))


def _full(shape):
    return pl.BlockSpec(shape, lambda i: (0,) * len(shape))


def _call(body, ins, outs):
    return pl.pallas_call(
        body,
        out_shape=outs,
        grid=(1,),
        in_specs=[_full(a.shape) for a in ins],
        out_specs=jax.tree.map(lambda s: _full(s.shape), outs),
    )(*ins)


def _unet_body(adj_ref, x_ref, w0_ref, w1_ref, w2_ref, w3_ref,
               b0_ref, b1_ref, b2_ref, b3_ref, p0_ref, p1_ref, p2_ref,
               u0_ref, u1_ref, u2_ref, c0_ref, c1_ref, c2_ref,
               o_ref, *, ks):
    k1, k2, k3 = ks
    adj = adj_ref[...]
    h0 = _gcn_t(adj, x_ref[...], w0_ref[...], b0_ref[...], relu=True)
    s0 = _score(h0, p0_ref[...])
    pt1 = _rank_pt(_rank(s0), k1)
    at1 = _pooled_adj(adj, pt1, transposed_in=True, exact_sel=False)
    h1 = _gcn(at1, _tdotf(pt1, h0) * _tdotf(pt1, s0),
              w1_ref[...], b1_ref[...], relu=True)
    s1 = _score(h1, p1_ref[...])
    pt2 = _rank_pt(_rank(s1), k2)
    at2 = _pooled_adj(at1, pt2, transposed_in=False, exact_sel=True)
    h2 = _gcn(at2, _tdotf(pt2, h1) * _tdotf(pt2, s1),
              w2_ref[...], b2_ref[...], relu=True)
    s2 = _score(h2, p2_ref[...])
    pt3 = _rank_pt(_rank(s2), k3)
    at3 = _pooled_adj(at2, pt3, transposed_in=False, exact_sel=True)
    x3 = _gcn(at3, _tdotf(pt3, h2) * _tdotf(pt3, s2),
              w3_ref[...], b3_ref[...], relu=True)
    g2 = _gcn(at2, h2 + _dotf(pt3, x3), u0_ref[...], c0_ref[...], relu=True)
    g1 = _gcn(at1, h1 + _dotf(pt2, g2), u1_ref[...], c1_ref[...], relu=True)
    o_ref[...] = _gcn_t(adj, h0 + _dotf(pt1, g1), u2_ref[...], c2_ref[...],
                        relu=False)


def kernel(x, adj, down_w_0, down_w_1, down_w_2, down_w_3,
           down_b_0, down_b_1, down_b_2, down_b_3,
           pool_p_0, pool_p_1, pool_p_2,
           up_w_0, up_w_1, up_w_2,
           up_b_0, up_b_1, up_b_2):
    n, co = adj.shape[0], up_w_2.shape[1]
    k1 = int(math.ceil(0.5 * n))
    k2 = int(math.ceil(0.5 * k1))
    k3 = int(math.ceil(0.5 * k2))
    ins = (adj, x, down_w_0, down_w_1, down_w_2, down_w_3,
           down_b_0, down_b_1, down_b_2, down_b_3,
           pool_p_0, pool_p_1, pool_p_2,
           up_w_0, up_w_1, up_w_2, up_b_0, up_b_1, up_b_2)
    return _call(functools.partial(_unet_body, ks=(k1, k2, k3)), ins,
                 jax.ShapeDtypeStruct((n, co), jnp.float32))
